# Initial kernel scaffold; baseline (speedup 1.0000x reference)
import jax
import jax.numpy as jnp
from jax.experimental import pallas as pl


def kernel(x, W1, b1, g1, bt1, Wl1, bl1, W2, b2, g2, bt2, Wl2, bl2, W3, b3, g3, bt3, Wl3, bl3, Wm, bm, edge_index):
    raise NotImplementedError("write your pallas kernel here")


# speedup vs baseline: 2.7416x; 2.7416x over previous
import functools

import jax
import jax.numpy as jnp
from jax import lax
from jax.experimental import pallas as pl
from jax.experimental.pallas import tpu as pltpu
from jax.experimental.pallas import tpu_sc as plsc

CW = 128
BN = 1000
NC = 2
NS = 16
LANES = 16
EPS = 1e-5

f32 = jnp.float32


def _mm_body(C, h_ref, a_ref, w_ref, b_ref, wl_ref, bl_ref, z_ref, s_ref, st_ref):
    i = pl.program_id(0)
    c = pl.program_id(1)

    @pl.when(c == 0)
    def _():
        z_ref[...] = jnp.broadcast_to(b_ref[...], z_ref.shape)
        s_ref[...] = jnp.broadcast_to(bl_ref[...], s_ref.shape)

    hb = h_ref[0]
    z_ref[...] += jnp.dot(hb + a_ref[0], w_ref[0], preferred_element_type=f32)
    s_ref[...] += jnp.dot(hb, wl_ref[0], preferred_element_type=f32)

    @pl.when((c == C - 1) & (i == 0))
    def _():
        st_ref[...] = jnp.zeros_like(st_ref)

    @pl.when(c == C - 1)
    def _():
        z = z_ref[...]
        st_ref[...] += jnp.concatenate(
            [jnp.sum(z, 0, keepdims=True),
             jnp.sum(z * z, 0, keepdims=True),
             jnp.zeros((6, z.shape[1]), f32)], axis=0)


def _mm_stats(h_ch, agg_ch, W, b, Wl, bl):
    C, N, _ = h_ch.shape
    H = W.shape[1]
    nb = N // BN
    Wr = W.reshape(C, CW, H)
    Wlr = Wl.reshape(C, CW, H)
    z, sk, st = pl.pallas_call(
        functools.partial(_mm_body, C),
        grid=(nb, C),
        in_specs=[
            pl.BlockSpec((1, BN, CW), lambda i, c: (c, i, 0)),
            pl.BlockSpec((1, BN, CW), lambda i, c: (c, i, 0)),
            pl.BlockSpec((1, CW, H), lambda i, c: (c, 0, 0)),
            pl.BlockSpec((1, H), lambda i, c: (0, 0)),
            pl.BlockSpec((1, CW, H), lambda i, c: (c, 0, 0)),
            pl.BlockSpec((1, H), lambda i, c: (0, 0)),
        ],
        out_specs=[
            pl.BlockSpec((BN, H), lambda i, c: (i, 0)),
            pl.BlockSpec((BN, H), lambda i, c: (i, 0)),
            pl.BlockSpec((8, H), lambda i, c: (0, 0)),
        ],
        out_shape=[
            jax.ShapeDtypeStruct((N, H), f32),
            jax.ShapeDtypeStruct((N, H), f32),
            jax.ShapeDtypeStruct((8, H), f32),
        ],
    )(h_ch, agg_ch, Wr, b.reshape(1, H), Wlr, bl.reshape(1, H))
    return z, sk, st


def _bn_body(N, act, z_ref, sk_ref, st_ref, g_ref, bt_ref, h_ref):
    s0 = st_ref[0]
    s1 = st_ref[1]
    mu = s0 * (1.0 / N)
    var = s1 * (1.0 / N) - mu * mu
    inv = lax.rsqrt(var + EPS)
    zn = (z_ref[...] - mu) * (inv * g_ref[0]) + bt_ref[0]
    a = jnp.maximum(zn, 0.0) + sk_ref[...]
    if act:
        a = jnp.where(a > 0.0, a, jnp.exp(a) - 1.0)
    h_ref[0] = a


def _bn_act(z, sk, st, g, bt, act):
    N, H = z.shape
    C = H // CW
    nb = N // BN
    return pl.pallas_call(
        functools.partial(_bn_body, N, act),
        grid=(nb, C),
        in_specs=[
            pl.BlockSpec((BN, CW), lambda i, c: (i, c)),
            pl.BlockSpec((BN, CW), lambda i, c: (i, c)),
            pl.BlockSpec((8, CW), lambda i, c: (0, c)),
            pl.BlockSpec((1, CW), lambda i, c: (0, c)),
            pl.BlockSpec((1, CW), lambda i, c: (0, c)),
        ],
        out_specs=pl.BlockSpec((1, BN, CW), lambda i, c: (c, i, 0)),
        out_shape=jax.ShapeDtypeStruct((C, N, CW), f32),
    )(z, sk, st, g.reshape(1, H), bt.reshape(1, H))


def _bn_s_body(N, C, z_ref, sk_ref, st_ref, g_ref, bt_ref, wm_ref, bm_ref, s_ref):
    c = pl.program_id(1)
    s0 = st_ref[0]
    s1 = st_ref[1]
    mu = s0 * (1.0 / N)
    var = s1 * (1.0 / N) - mu * mu
    inv = lax.rsqrt(var + EPS)
    zn = (z_ref[...] - mu) * (inv * g_ref[0]) + bt_ref[0]
    a = jnp.maximum(zn, 0.0) + sk_ref[...]

    @pl.when(c == 0)
    def _():
        s_ref[...] = jnp.broadcast_to(bm_ref[...], s_ref.shape)

    s_ref[...] += jnp.dot(a, wm_ref[0], preferred_element_type=f32)


def _bn_scores(z, sk, st, g, bt, Wm, bm):
    N, H = z.shape
    C = H // CW
    nb = N // BN
    wm2 = jnp.concatenate([Wm[:H], Wm[H:]], axis=1).reshape(C, CW, 2)
    bm2 = jnp.stack([bm[0], jnp.zeros((), f32)]).reshape(1, 2)
    return pl.pallas_call(
        functools.partial(_bn_s_body, N, C),
        grid=(nb, C),
        in_specs=[
            pl.BlockSpec((BN, CW), lambda i, c: (i, c)),
            pl.BlockSpec((BN, CW), lambda i, c: (i, c)),
            pl.BlockSpec((8, CW), lambda i, c: (0, c)),
            pl.BlockSpec((1, CW), lambda i, c: (0, c)),
            pl.BlockSpec((1, CW), lambda i, c: (0, c)),
            pl.BlockSpec((1, CW, 2), lambda i, c: (c, 0, 0)),
            pl.BlockSpec((1, 2), lambda i, c: (0, 0)),
        ],
        out_specs=pl.BlockSpec((BN, 2), lambda i, c: (i, 0)),
        out_shape=jax.ShapeDtypeStruct((N, 2), f32),
    )(z, sk, st, g.reshape(1, H), bt.reshape(1, H), wm2, bm2)


@functools.lru_cache(maxsize=None)
def _make_segsum(C, N_pad, NB, B, KB, SEG):
    RPT = N_pad // NS
    NBS = NB // SEG
    GRP = NBS // KB
    mesh = plsc.VectorSubcoreMesh(core_axis_name="c", subcore_axis_name="s",
                                  num_cores=NC, num_subcores=NS)

    @functools.partial(
        pl.kernel,
        out_type=jax.ShapeDtypeStruct((C, N_pad, CW), f32),
        mesh=mesh,
        scratch_types=[
            pltpu.VMEM((NBS, B), jnp.int32),
            pltpu.VMEM((NBS, B), jnp.int32),
            pltpu.VMEM((KB, B, CW), f32),
            pltpu.VMEM_SHARED((N_pad, CW), f32),
            pltpu.SemaphoreType.DMA,
            pltpu.SemaphoreType.DMA,
        ],
    )
    def seg(h_hbm, rowp_hbm, colp_hbm, zeros_hbm, out_hbm,
            row_v, col_v, gbuf, shared, gsem, ssem):
        ci = lax.axis_index("c")
        t = lax.axis_index("s")
        for c in range(C):
            @pl.when(ci == (c % NC))
            def _(c=c):
                pltpu.sync_copy(zeros_hbm, shared.at[pl.ds(t * RPT, RPT)])
                plsc.subcore_barrier()

                for s in range(SEG):
                    pltpu.sync_copy(
                        rowp_hbm.at[t].at[pl.ds(s * NBS, NBS)], row_v)
                    pltpu.sync_copy(
                        colp_hbm.at[t].at[pl.ds(s * NBS, NBS)], col_v)

                    def grp(g, carry):
                        hnds = []
                        for b in range(KB):
                            j = g * KB + b
                            hnds.append(pltpu.async_copy(
                                h_hbm.at[c].at[row_v.at[j]], gbuf.at[b], gsem))
                        for hd in hnds:
                            hd.wait()
                        snds = []
                        for b in range(KB):
                            j = g * KB + b
                            snds.append(pltpu.async_copy(
                                gbuf.at[b], shared.at[col_v.at[j]], ssem,
                                add=True))
                        for sd in snds:
                            sd.wait()
                        return carry

                    lax.fori_loop(0, GRP, grp, 0)
                plsc.subcore_barrier()
                for p in range(RPT // B):
                    pltpu.sync_copy(shared.at[pl.ds(t * RPT + p * B, B)],
                                    gbuf.at[0])
                    pltpu.sync_copy(gbuf.at[0],
                                    out_hbm.at[c].at[pl.ds(t * RPT + p * B, B)])
                plsc.subcore_barrier()

    return seg


@functools.lru_cache(maxsize=None)
def _make_edge(N, EP):
    NIT = EP // LANES
    mesh = plsc.VectorSubcoreMesh(core_axis_name="c", subcore_axis_name="s",
                                  num_cores=NC, num_subcores=NS)

    @functools.partial(
        pl.kernel,
        out_type=jax.ShapeDtypeStruct((NC * NS * EP,), f32),
        mesh=mesh,
        compiler_params=pltpu.CompilerParams(needs_layout_passes=False),
        scratch_types=[
            pltpu.VMEM((N,), f32),
            pltpu.VMEM((N,), f32),
            pltpu.VMEM((EP,), jnp.int32),
            pltpu.VMEM((EP,), jnp.int32),
            pltpu.VMEM((EP,), f32),
        ],
    )
    def edge(sa_hbm, sb_hbm, row_hbm, col_hbm, out_hbm,
             sa_v, sb_v, row_v, col_v, o_v):
        w = lax.axis_index("s") * NC + lax.axis_index("c")
        base = w * EP
        pltpu.sync_copy(sa_hbm, sa_v)
        pltpu.sync_copy(sb_hbm, sb_v)
        pltpu.sync_copy(row_hbm.at[pl.ds(base, EP)], row_v)
        pltpu.sync_copy(col_hbm.at[pl.ds(base, EP)], col_v)

        def it(j, carry):
            r = row_v[pl.ds(j * LANES, LANES)]
            cc = col_v[pl.ds(j * LANES, LANES)]
            va = plsc.load_gather(sa_v, [r])
            vb = plsc.load_gather(sb_v, [cc])
            x = va + vb
            o_v[pl.ds(j * LANES, LANES)] = 1.0 / (1.0 + jnp.exp(-x))
            return carry

        lax.fori_loop(0, NIT, it, 0)
        pltpu.sync_copy(o_v, out_hbm.at[pl.ds(base, EP)])

    return edge


def _ceil_to(a, m):
    return (a + m - 1) // m * m


def kernel(x, W1, b1, g1, bt1, Wl1, bl1, W2, b2, g2, bt2, Wl2, bl2,
           W3, b3, g3, bt3, Wl3, bl3, Wm, bm, edge_index):
    N, F = x.shape
    H = W1.shape[1]
    E = edge_index.shape[1]
    row = edge_index[0]
    col = edge_index[1]

    B = 128
    KB = 2
    SEG = 2
    N_pad = _ceil_to(N, NS * B)
    per_tile = _ceil_to(E // NS + (1 if E % NS else 0), B * KB * SEG)
    NB = per_tile // B
    E_pad = NS * per_tile
    rowp = jnp.pad(row, (0, E_pad - E)).reshape(NS, NB, B)
    colp = jnp.pad(col, (0, E_pad - E), constant_values=N).reshape(NS, NB, B)
    zeros = jnp.zeros((N_pad // NS, CW), f32)

    x_ch = x.reshape(N, F // CW, CW).transpose(1, 0, 2)

    seg_a = _make_segsum(F // CW, N_pad, NB, B, KB, SEG)
    seg_h = _make_segsum(H // CW, N_pad, NB, B, KB, SEG)

    agg1 = seg_a(x_ch, rowp, colp, zeros)
    z1, sk1, st1 = _mm_stats(x_ch, agg1, W1, b1, Wl1, bl1)
    h1 = _bn_act(z1, sk1, st1, g1, bt1, act=True)
    agg2 = seg_h(h1, rowp, colp, zeros)
    z2, sk2, st2 = _mm_stats(h1, agg2, W2, b2, Wl2, bl2)
    h2 = _bn_act(z2, sk2, st2, g2, bt2, act=True)
    agg3 = seg_h(h2, rowp, colp, zeros)
    z3, sk3, st3 = _mm_stats(h2, agg3, W3, b3, Wl3, bl3)
    s = _bn_scores(z3, sk3, st3, g3, bt3, Wm, bm)

    EP = _ceil_to(E, NC * NS * LANES) // (NC * NS)
    E_pad2 = NC * NS * EP
    row2 = jnp.pad(row, (0, E_pad2 - E))
    col2 = jnp.pad(col, (0, E_pad2 - E))
    edge_k = _make_edge(N, EP)
    out = edge_k(s[:, 0], s[:, 1], row2, col2)
    return out[:E]


# speedup vs baseline: 3.0757x; 1.1219x over previous
import functools

import jax
import jax.numpy as jnp
from jax import lax
from jax.experimental import pallas as pl
from jax.experimental.pallas import tpu as pltpu
from jax.experimental.pallas import tpu_sc as plsc

CW = 128
BN = 1000
NC = 2
NS = 16
LANES = 16
EPS = 1e-5

f32 = jnp.float32


def _mm_body(C, h_ref, a_ref, w_ref, b_ref, wl_ref, bl_ref, z_ref, s_ref, st_ref):
    i = pl.program_id(0)
    c = pl.program_id(1)

    @pl.when(c == 0)
    def _():
        z_ref[...] = jnp.broadcast_to(b_ref[...], z_ref.shape)
        s_ref[...] = jnp.broadcast_to(bl_ref[...], s_ref.shape)

    hb = h_ref[0]
    z_ref[...] += jnp.dot(hb + a_ref[0], w_ref[0], preferred_element_type=f32)
    s_ref[...] += jnp.dot(hb, wl_ref[0], preferred_element_type=f32)

    @pl.when((c == C - 1) & (i == 0))
    def _():
        st_ref[...] = jnp.zeros_like(st_ref)

    @pl.when(c == C - 1)
    def _():
        z = z_ref[...]
        st_ref[...] += jnp.concatenate(
            [jnp.sum(z, 0, keepdims=True),
             jnp.sum(z * z, 0, keepdims=True),
             jnp.zeros((6, z.shape[1]), f32)], axis=0)


def _mm_stats(h_ch, agg_ch, W, b, Wl, bl):
    C, N, _ = h_ch.shape
    H = W.shape[1]
    nb = N // BN
    Wr = W.reshape(C, CW, H)
    Wlr = Wl.reshape(C, CW, H)
    z, sk, st = pl.pallas_call(
        functools.partial(_mm_body, C),
        grid=(nb, C),
        in_specs=[
            pl.BlockSpec((1, BN, CW), lambda i, c: (c, i, 0)),
            pl.BlockSpec((1, BN, CW), lambda i, c: (c, i, 0)),
            pl.BlockSpec((1, CW, H), lambda i, c: (c, 0, 0)),
            pl.BlockSpec((1, H), lambda i, c: (0, 0)),
            pl.BlockSpec((1, CW, H), lambda i, c: (c, 0, 0)),
            pl.BlockSpec((1, H), lambda i, c: (0, 0)),
        ],
        out_specs=[
            pl.BlockSpec((BN, H), lambda i, c: (i, 0)),
            pl.BlockSpec((BN, H), lambda i, c: (i, 0)),
            pl.BlockSpec((8, H), lambda i, c: (0, 0)),
        ],
        out_shape=[
            jax.ShapeDtypeStruct((N, H), f32),
            jax.ShapeDtypeStruct((N, H), f32),
            jax.ShapeDtypeStruct((8, H), f32),
        ],
    )(h_ch, agg_ch, Wr, b.reshape(1, H), Wlr, bl.reshape(1, H))
    return z, sk, st


def _bn_body(N, act, z_ref, sk_ref, st_ref, g_ref, bt_ref, h_ref):
    s0 = st_ref[0]
    s1 = st_ref[1]
    mu = s0 * (1.0 / N)
    var = s1 * (1.0 / N) - mu * mu
    inv = lax.rsqrt(var + EPS)
    zn = (z_ref[...] - mu) * (inv * g_ref[0]) + bt_ref[0]
    a = jnp.maximum(zn, 0.0) + sk_ref[...]
    if act:
        a = jnp.where(a > 0.0, a, jnp.exp(a) - 1.0)
    h_ref[0] = a


def _bn_act(z, sk, st, g, bt, act):
    N, H = z.shape
    C = H // CW
    nb = N // BN
    return pl.pallas_call(
        functools.partial(_bn_body, N, act),
        grid=(nb, C),
        in_specs=[
            pl.BlockSpec((BN, CW), lambda i, c: (i, c)),
            pl.BlockSpec((BN, CW), lambda i, c: (i, c)),
            pl.BlockSpec((8, CW), lambda i, c: (0, c)),
            pl.BlockSpec((1, CW), lambda i, c: (0, c)),
            pl.BlockSpec((1, CW), lambda i, c: (0, c)),
        ],
        out_specs=pl.BlockSpec((1, BN, CW), lambda i, c: (c, i, 0)),
        out_shape=jax.ShapeDtypeStruct((C, N, CW), f32),
    )(z, sk, st, g.reshape(1, H), bt.reshape(1, H))


def _bn_s_body(N, C, z_ref, sk_ref, st_ref, g_ref, bt_ref, wm_ref, bm_ref, s_ref):
    c = pl.program_id(1)
    s0 = st_ref[0]
    s1 = st_ref[1]
    mu = s0 * (1.0 / N)
    var = s1 * (1.0 / N) - mu * mu
    inv = lax.rsqrt(var + EPS)
    zn = (z_ref[...] - mu) * (inv * g_ref[0]) + bt_ref[0]
    a = jnp.maximum(zn, 0.0) + sk_ref[...]

    @pl.when(c == 0)
    def _():
        s_ref[...] = jnp.broadcast_to(bm_ref[...], s_ref.shape)

    s_ref[...] += jnp.dot(a, wm_ref[0], preferred_element_type=f32)


def _bn_scores(z, sk, st, g, bt, Wm, bm):
    N, H = z.shape
    C = H // CW
    nb = N // BN
    wm2 = jnp.concatenate([Wm[:H], Wm[H:]], axis=1).reshape(C, CW, 2)
    bm2 = jnp.stack([bm[0], jnp.zeros((), f32)]).reshape(1, 2)
    return pl.pallas_call(
        functools.partial(_bn_s_body, N, C),
        grid=(nb, C),
        in_specs=[
            pl.BlockSpec((BN, CW), lambda i, c: (i, c)),
            pl.BlockSpec((BN, CW), lambda i, c: (i, c)),
            pl.BlockSpec((8, CW), lambda i, c: (0, c)),
            pl.BlockSpec((1, CW), lambda i, c: (0, c)),
            pl.BlockSpec((1, CW), lambda i, c: (0, c)),
            pl.BlockSpec((1, CW, 2), lambda i, c: (c, 0, 0)),
            pl.BlockSpec((1, 2), lambda i, c: (0, 0)),
        ],
        out_specs=pl.BlockSpec((BN, 2), lambda i, c: (i, 0)),
        out_shape=jax.ShapeDtypeStruct((N, 2), f32),
    )(z, sk, st, g.reshape(1, H), bt.reshape(1, H), wm2, bm2)


@functools.lru_cache(maxsize=None)
def _make_segsum(C, N_pad, NB, B, KB, SEG):
    RPT = N_pad // NS
    NBS = NB // SEG
    GRP = NBS // KB
    mesh = plsc.VectorSubcoreMesh(core_axis_name="c", subcore_axis_name="s",
                                  num_cores=NC, num_subcores=NS)

    @functools.partial(
        pl.kernel,
        out_type=jax.ShapeDtypeStruct((C, N_pad, CW), f32),
        mesh=mesh,
        scratch_types=[
            pltpu.VMEM((NBS, B), jnp.int32),
            pltpu.VMEM((NBS, B), jnp.int32),
            pltpu.VMEM((2, B, CW), f32),
            pltpu.VMEM_SHARED((N_pad, CW), f32),
            pltpu.SemaphoreType.DMA,
            pltpu.SemaphoreType.DMA,
            pltpu.SemaphoreType.DMA,
            pltpu.SemaphoreType.DMA,
        ],
    )
    def seg(h_hbm, rowp_hbm, colp_hbm, zeros_hbm, out_hbm,
            row_v, col_v, gbuf, shared, gsem0, gsem1, ssem0, ssem1):
        ci = lax.axis_index("c")
        t = lax.axis_index("s")
        for c in range(C):
            @pl.when(ci == (c % NC))
            def _(c=c):
                gsem = (gsem0, gsem1)
                ssem = (ssem0, ssem1)

                def issue_g(j, p):
                    pltpu.async_copy(h_hbm.at[c].at[row_v.at[j]],
                                     gbuf.at[p], gsem[p])

                def wait_g(p):
                    pltpu.make_async_copy(h_hbm.at[c].at[row_v.at[0]],
                                          gbuf.at[p], gsem[p]).wait()

                def issue_s(j, p):
                    pltpu.async_copy(gbuf.at[p], shared.at[col_v.at[j]],
                                     ssem[p], add=True)

                def wait_s(p):
                    pltpu.make_async_copy(gbuf.at[p],
                                          shared.at[col_v.at[0]],
                                          ssem[p]).wait()

                pltpu.sync_copy(zeros_hbm, shared.at[pl.ds(t * RPT, RPT)])
                plsc.subcore_barrier()

                for s in range(SEG):
                    pltpu.sync_copy(
                        rowp_hbm.at[t].at[pl.ds(s * NBS, NBS)], row_v)
                    pltpu.sync_copy(
                        colp_hbm.at[t].at[pl.ds(s * NBS, NBS)], col_v)

                    issue_g(0, 0)
                    HALF = NBS // 2

                    def grp(i, carry):
                        j0 = 2 * i

                        @pl.when(i > 0)
                        def _():
                            wait_s(1)
                        issue_g(j0 + 1, 1)
                        wait_g(0)
                        issue_s(j0, 0)
                        wait_s(0)

                        @pl.when(i < HALF - 1)
                        def _():
                            issue_g(j0 + 2, 0)
                        wait_g(1)
                        issue_s(j0 + 1, 1)
                        return carry

                    lax.fori_loop(0, HALF, grp, 0)
                    wait_s(1)
                plsc.subcore_barrier()
                for p in range(RPT // B):
                    pltpu.sync_copy(shared.at[pl.ds(t * RPT + p * B, B)],
                                    gbuf.at[0])
                    pltpu.sync_copy(gbuf.at[0],
                                    out_hbm.at[c].at[pl.ds(t * RPT + p * B, B)])
                plsc.subcore_barrier()

    return seg


@functools.lru_cache(maxsize=None)
def _make_edge(N, EP):
    NIT = EP // LANES
    mesh = plsc.VectorSubcoreMesh(core_axis_name="c", subcore_axis_name="s",
                                  num_cores=NC, num_subcores=NS)

    @functools.partial(
        pl.kernel,
        out_type=jax.ShapeDtypeStruct((NC * NS * EP,), f32),
        mesh=mesh,
        compiler_params=pltpu.CompilerParams(needs_layout_passes=False),
        scratch_types=[
            pltpu.VMEM((N,), f32),
            pltpu.VMEM((N,), f32),
            pltpu.VMEM((EP,), jnp.int32),
            pltpu.VMEM((EP,), jnp.int32),
            pltpu.VMEM((EP,), f32),
        ],
    )
    def edge(sa_hbm, sb_hbm, row_hbm, col_hbm, out_hbm,
             sa_v, sb_v, row_v, col_v, o_v):
        w = lax.axis_index("s") * NC + lax.axis_index("c")
        base = w * EP
        pltpu.sync_copy(sa_hbm, sa_v)
        pltpu.sync_copy(sb_hbm, sb_v)
        pltpu.sync_copy(row_hbm.at[pl.ds(base, EP)], row_v)
        pltpu.sync_copy(col_hbm.at[pl.ds(base, EP)], col_v)

        def it(j, carry):
            r = row_v[pl.ds(j * LANES, LANES)]
            cc = col_v[pl.ds(j * LANES, LANES)]
            va = plsc.load_gather(sa_v, [r])
            vb = plsc.load_gather(sb_v, [cc])
            x = va + vb
            o_v[pl.ds(j * LANES, LANES)] = 1.0 / (1.0 + jnp.exp(-x))
            return carry

        lax.fori_loop(0, NIT, it, 0)
        pltpu.sync_copy(o_v, out_hbm.at[pl.ds(base, EP)])

    return edge


def _ceil_to(a, m):
    return (a + m - 1) // m * m


def kernel(x, W1, b1, g1, bt1, Wl1, bl1, W2, b2, g2, bt2, Wl2, bl2,
           W3, b3, g3, bt3, Wl3, bl3, Wm, bm, edge_index):
    N, F = x.shape
    H = W1.shape[1]
    E = edge_index.shape[1]
    row = edge_index[0]
    col = edge_index[1]

    B = 128
    KB = 2
    SEG = 2
    N_pad = _ceil_to(N, NS * B)
    per_tile = _ceil_to(E // NS + (1 if E % NS else 0), B * KB * SEG)
    NB = per_tile // B
    E_pad = NS * per_tile
    rowp = jnp.pad(row, (0, E_pad - E)).reshape(NS, NB, B)
    colp = jnp.pad(col, (0, E_pad - E), constant_values=N).reshape(NS, NB, B)
    zeros = jnp.zeros((N_pad // NS, CW), f32)

    x_ch = x.reshape(N, F // CW, CW).transpose(1, 0, 2)

    seg_a = _make_segsum(F // CW, N_pad, NB, B, KB, SEG)
    seg_h = _make_segsum(H // CW, N_pad, NB, B, KB, SEG)

    agg1 = seg_a(x_ch, rowp, colp, zeros)
    z1, sk1, st1 = _mm_stats(x_ch, agg1, W1, b1, Wl1, bl1)
    h1 = _bn_act(z1, sk1, st1, g1, bt1, act=True)
    agg2 = seg_h(h1, rowp, colp, zeros)
    z2, sk2, st2 = _mm_stats(h1, agg2, W2, b2, Wl2, bl2)
    h2 = _bn_act(z2, sk2, st2, g2, bt2, act=True)
    agg3 = seg_h(h2, rowp, colp, zeros)
    z3, sk3, st3 = _mm_stats(h2, agg3, W3, b3, Wl3, bl3)
    s = _bn_scores(z3, sk3, st3, g3, bt3, Wm, bm)

    EP = _ceil_to(E, NC * NS * LANES) // (NC * NS)
    E_pad2 = NC * NS * EP
    row2 = jnp.pad(row, (0, E_pad2 - E))
    col2 = jnp.pad(col, (0, E_pad2 - E))
    edge_k = _make_edge(N, EP)
    out = edge_k(s[:, 0], s[:, 1], row2, col2)
    return out[:E]
